# Initial kernel scaffold; baseline (speedup 1.0000x reference)
import jax
import jax.numpy as jnp
from jax.experimental import pallas as pl


def kernel(images, patch_w, patch_b, cls_token, pos_embed, n1g, n1b, qkv_w, qkv_b, proj_w, proj_b, n2g, n2b, fc1_w, fc1_b, fc2_w, fc2_b, norm_g, norm_b, head_w, head_b):
    raise NotImplementedError("write your pallas kernel here")


# speedup vs baseline: 1.7535x; 1.7535x over previous
import math

import jax
import jax.numpy as jnp
from jax.experimental import pallas as pl

_B = 8
_IMG = 224
_PATCH = 16
_GRID = _IMG // _PATCH
_NPATCH = _GRID * _GRID
_S = _NPATCH + 1
_D = 768
_H = 12
_HD = _D // _H
_DEPTH = 12
_PRUNE_AFTER = 4
_MLP = 4 * _D
_NCLS = 1000
_NKEEP = 98
_KP = 104
_PDIM = 3 * _PATCH * _PATCH
_EPS = 1e-6
_ISQRT2 = 0.7071067811865476
_ASCALE = 1.0 / math.sqrt(_HD)


def _mm(a, b):
    return jax.lax.dot_general(a, b, (((1,), (0,)), ((), ())),
                               preferred_element_type=jnp.float32)


def _mmT(a, b):
    return jax.lax.dot_general(a, b, (((1,), (1,)), ((), ())),
                               preferred_element_type=jnp.float32)


def _ln(x, g, b):
    m = jnp.mean(x, axis=-1, keepdims=True)
    xc = x - m
    v = jnp.mean(xc * xc, axis=-1, keepdims=True)
    return xc * jax.lax.rsqrt(v + _EPS) * g + b


def _gelu(x):
    return 0.5 * x * (1.0 + jax.lax.erf(x * _ISQRT2))


def _front_kernel(p_ref, pw_ref, pb_ref, cls_ref, pos0_ref, posr_ref, out_ref):
    emb = _mmT(p_ref[0], pw_ref[...]) + pb_ref[...] + posr_ref[...]
    row0 = cls_ref[...] + pos0_ref[...]
    out_ref[0] = jnp.concatenate([row0, emb], axis=0)


def _front(p, patch_w, patch_b, cls_tok, pos0, posr):
    return pl.pallas_call(
        _front_kernel,
        grid=(_B,),
        in_specs=[
            pl.BlockSpec((1, _NPATCH, _PDIM), lambda b: (b, 0, 0)),
            pl.BlockSpec((_D, _PDIM), lambda b: (0, 0)),
            pl.BlockSpec((1, _D), lambda b: (0, 0)),
            pl.BlockSpec((1, _D), lambda b: (0, 0)),
            pl.BlockSpec((1, _D), lambda b: (0, 0)),
            pl.BlockSpec((_NPATCH, _D), lambda b: (0, 0)),
        ],
        out_specs=pl.BlockSpec((1, _S, _D), lambda b: (b, 0, 0)),
        out_shape=jax.ShapeDtypeStruct((_B, _S, _D), jnp.float32),
    )(p, patch_w, patch_b, cls_tok, pos0, posr)


def _block_kernel(x_ref, m_ref, g1_ref, b1_ref, qw_ref, qb_ref, pw_ref, pb_ref,
                  g2_ref, b2_ref, w1_ref, c1_ref, w2_ref, c2_ref, out_ref):
    x = x_ref[0]
    h = _ln(x, g1_ref[0], b1_ref[0])
    qkv = _mmT(h, qw_ref[0]) + qb_ref[0]
    neg = (1.0 - m_ref[0]) * (-1e30)
    outs = []
    for i in range(_H):
        qh = qkv[:, i * _HD:(i + 1) * _HD]
        kh = qkv[:, _D + i * _HD:_D + (i + 1) * _HD]
        vh = qkv[:, 2 * _D + i * _HD:2 * _D + (i + 1) * _HD]
        l = _mmT(qh, kh) * _ASCALE + neg
        l = l - jnp.max(l, axis=-1, keepdims=True)
        e = jnp.exp(l)
        a = e / jnp.sum(e, axis=-1, keepdims=True)
        outs.append(_mm(a, vh))
    o = jnp.concatenate(outs, axis=1)
    x = x + _mmT(o, pw_ref[0]) + pb_ref[0]
    h2 = _ln(x, g2_ref[0], b2_ref[0])
    mh = _gelu(_mmT(h2, w1_ref[0]) + c1_ref[0])
    out_ref[0] = x + _mmT(mh, w2_ref[0]) + c2_ref[0]


def _run_block(x, mask, i, n1g, n1b, qkv_w, qkv_b, proj_w, proj_b,
               n2g, n2b, fc1_w, fc1_b, fc2_w, fc2_b):
    seq = x.shape[1]

    def w3(shape):
        return pl.BlockSpec((1,) + shape, lambda b: (i, 0, 0))

    return pl.pallas_call(
        _block_kernel,
        grid=(_B,),
        in_specs=[
            pl.BlockSpec((1, seq, _D), lambda b: (b, 0, 0)),
            pl.BlockSpec((1, 1, seq), lambda b: (b, 0, 0)),
            w3((1, _D)), w3((1, _D)),
            w3((3 * _D, _D)), w3((1, 3 * _D)),
            w3((_D, _D)), w3((1, _D)),
            w3((1, _D)), w3((1, _D)),
            w3((_MLP, _D)), w3((1, _MLP)),
            w3((_D, _MLP)), w3((1, _D)),
        ],
        out_specs=pl.BlockSpec((1, seq, _D), lambda b: (b, 0, 0)),
        out_shape=jax.ShapeDtypeStruct((_B, seq, _D), jnp.float32),
    )(x, mask, n1g, n1b, qkv_w, qkv_b, proj_w, proj_b,
      n2g, n2b, fc1_w, fc1_b, fc2_w, fc2_b)


def _transpose_col(col, n):
    i_col = jax.lax.broadcasted_iota(jnp.int32, (n, 1), 0)
    j_row = jax.lax.broadcasted_iota(jnp.int32, (1, n), 1)
    return jnp.sum(jnp.where(i_col == j_row, col, 0.0), axis=0, keepdims=True)


def _pack_kernel(x_ref, out_ref, valid_ref):
    x = x_ref[0]
    xx = x * x
    s_col = jnp.sqrt(_mm(xx, jnp.ones((_D, 1), jnp.float32)))
    s_row = _transpose_col(s_col, _S)
    i_col = jax.lax.broadcasted_iota(jnp.int32, (_S, 1), 0)
    j_row = jax.lax.broadcasted_iota(jnp.int32, (1, _S), 1)
    beats = (s_row > s_col) | ((s_row == s_col) & (j_row < i_col))
    rank = jnp.sum(beats.astype(jnp.float32), axis=1, keepdims=True)
    keep_col = ((rank < float(_NKEEP)) | (i_col == 0)).astype(jnp.float32)
    keep_row = _transpose_col(keep_col, _S)
    count = jnp.sum(keep_col)
    below = (j_row < i_col).astype(jnp.float32)
    pos_col = jnp.sum(keep_row * below, axis=1, keepdims=True)
    pos_row = _transpose_col(pos_col, _S)
    p_col = jax.lax.broadcasted_iota(jnp.int32, (_KP, 1), 0).astype(jnp.float32)
    sel = ((p_col == pos_row) & (keep_row > 0.5)).astype(jnp.float32)
    out_ref[0] = _mm(sel, x)
    kp_row = jax.lax.broadcasted_iota(jnp.int32, (1, _KP), 1).astype(jnp.float32)
    valid_ref[0] = (kp_row < count).astype(jnp.float32)


def _pack(x):
    return pl.pallas_call(
        _pack_kernel,
        grid=(_B,),
        in_specs=[pl.BlockSpec((1, _S, _D), lambda b: (b, 0, 0))],
        out_specs=[
            pl.BlockSpec((1, _KP, _D), lambda b: (b, 0, 0)),
            pl.BlockSpec((1, 1, _KP), lambda b: (b, 0, 0)),
        ],
        out_shape=[
            jax.ShapeDtypeStruct((_B, _KP, _D), jnp.float32),
            jax.ShapeDtypeStruct((_B, 1, _KP), jnp.float32),
        ],
    )(x)


def _head_kernel(x_ref, g_ref, b_ref, w_ref, hb_ref, out_ref):
    h = _ln(x_ref[...], g_ref[...], b_ref[...])
    out_ref[...] = _mmT(h, w_ref[...]) + hb_ref[...]


def _head(cls, norm_g, norm_b, head_w, head_b):
    return pl.pallas_call(
        _head_kernel,
        out_shape=jax.ShapeDtypeStruct((_B, _NCLS), jnp.float32),
    )(cls, norm_g, norm_b, head_w, head_b)


def kernel(images, patch_w, patch_b, cls_token, pos_embed, n1g, n1b, qkv_w,
           qkv_b, proj_w, proj_b, n2g, n2b, fc1_w, fc1_b, fc2_w, fc2_b,
           norm_g, norm_b, head_w, head_b):
    p = images.reshape(_B, 3, _GRID, _PATCH, _GRID, _PATCH)
    p = p.transpose(0, 2, 4, 1, 3, 5).reshape(_B, _NPATCH, _PDIM)
    pos = pos_embed.reshape(_S, _D)
    x = _front(p, patch_w, patch_b.reshape(1, _D), cls_token.reshape(1, _D),
               pos[0:1], pos[1:])

    n1g3 = n1g.reshape(_DEPTH, 1, _D)
    n1b3 = n1b.reshape(_DEPTH, 1, _D)
    qkv_b3 = qkv_b.reshape(_DEPTH, 1, 3 * _D)
    proj_b3 = proj_b.reshape(_DEPTH, 1, _D)
    n2g3 = n2g.reshape(_DEPTH, 1, _D)
    n2b3 = n2b.reshape(_DEPTH, 1, _D)
    fc1_b3 = fc1_b.reshape(_DEPTH, 1, _MLP)
    fc2_b3 = fc2_b.reshape(_DEPTH, 1, _D)

    def layer(xx, mask, i):
        return _run_block(xx, mask, i, n1g3, n1b3, qkv_w, qkv_b3, proj_w,
                          proj_b3, n2g3, n2b3, fc1_w, fc1_b3, fc2_w, fc2_b3)

    dense_mask = jnp.ones((_B, 1, _S), jnp.float32)
    for i in range(_PRUNE_AFTER):
        x = layer(x, dense_mask, i)

    packed, valid = _pack(x)
    for i in range(_PRUNE_AFTER, _DEPTH):
        packed = layer(packed, valid, i)

    cls = packed[:, 0, :]
    return _head(cls, norm_g.reshape(1, _D), norm_b.reshape(1, _D),
                 head_w, head_b.reshape(1, _NCLS))
